# baseline (speedup 1.0000x reference)
import functools
import math

import jax
import jax.numpy as jnp
import numpy as np
from jax import lax
from jax.experimental import pallas as pl
from jax.experimental.pallas import tpu as pltpu
from jax.experimental.pallas import tpu_sc as plsc

B, L, D = 16, 2048, 128
V = 100000
K = 30
KP = 32
D_OUT = D + (K * (K + 1)) // 2
LANES = 16
NCHUNK = L // LANES
GATHER_ROWS = 16
GATHER_W = L // GATHER_ROWS

_IU_I, _IU_J = np.triu_indices(K)


def _sc_body(ids_hbm, pm_hbm, tw_hbm, t2c_hbm, w_out, seg_out,
             idsf, pmf, twgf, clgf, table, wf, segf, sem_tw, sem_cl):
  wid = lax.axis_index("s") * 2 + lax.axis_index("c")

  @pl.when(wid < B)
  def _():
    b = wid
    pltpu.sync_copy(ids_hbm.at[b], idsf)
    pltpu.sync_copy(pm_hbm.at[b], pmf)

    copies = []
    for j in range(GATHER_ROWS):
      sl = pl.ds(j * GATHER_W, GATHER_W)
      copies.append(pltpu.async_copy(
          tw_hbm.at[idsf.at[sl]], twgf.at[sl], sem_tw))
      copies.append(pltpu.async_copy(
          t2c_hbm.at[idsf.at[sl]], clgf.at[sl], sem_cl))

    lane = lax.iota(jnp.int32, LANES)

    def scatter_body(c, _):
      sl = pl.ds(c * LANES, LANES)
      id16 = idsf[sl]
      valid = pmf[sl] != 0
      pos16 = c * LANES + lane
      _, lastm = plsc.scan_count(id16, mask=valid)
      plsc.store_scatter(table, [id16], pos16, mask=lastm & valid)
      return 0

    lax.fori_loop(0, NCHUNK, scatter_body, 0, unroll=4)

    for cp in copies:
      cp.wait()

    def check_body(c, _):
      sl = pl.ds(c * LANES, LANES)
      id16 = idsf[sl]
      valid = pmf[sl] != 0
      pos16 = c * LANES + lane
      lastpos = plsc.load_gather(table, [id16])
      is_last = valid & (lastpos == pos16)
      wf[sl] = jnp.where(is_last, twgf[sl], 0.0)
      segf[sl] = jnp.where(is_last, clgf[sl], K)
      return 0

    lax.fori_loop(0, NCHUNK, check_body, 0, unroll=4)

    pltpu.sync_copy(wf, w_out.at[b])
    pltpu.sync_copy(segf, seg_out.at[b])


def _sc_tokens(input_ids, padding_mask, token_weights, token_to_cluster):
  return pl.kernel(
      _sc_body,
      out_type=(
          jax.ShapeDtypeStruct((B, L), jnp.float32),
          jax.ShapeDtypeStruct((B, L), jnp.int32),
      ),
      mesh=plsc.VectorSubcoreMesh(core_axis_name="c", subcore_axis_name="s"),
      compiler_params=pltpu.CompilerParams(needs_layout_passes=False),
      scratch_types=[
          pltpu.VMEM((L,), jnp.int32),
          pltpu.VMEM((L,), jnp.int32),
          pltpu.VMEM((L,), jnp.float32),
          pltpu.VMEM((L,), jnp.int32),
          pltpu.VMEM((V,), jnp.int32),
          pltpu.VMEM((L,), jnp.float32),
          pltpu.VMEM((L,), jnp.int32),
          pltpu.SemaphoreType.DMA,
          pltpu.SemaphoreType.DMA,
      ],
  )(input_ids, padding_mask, token_weights, token_to_cluster)


NT = (K * (K + 1)) // 2

_RI = np.zeros((NT, KP), np.float32)
_RI[np.arange(NT), _IU_I] = np.where(_IU_I == _IU_J, 1.0, math.sqrt(2.0))
_RJ = np.zeros((NT, KP), np.float32)
_RJ[np.arange(NT), _IU_J] = 1.0


NB = 8


def _tc_one_batch(seq, w, seg, cent_ref, ri_ref, rj_ref, svd_ref):
  kidx = lax.broadcasted_iota(jnp.int32, (KP, L), 0)
  oh = seg == kidx
  ohw = jnp.where(oh, w, 0.0)
  m = (seg < K).astype(jnp.float32)
  ohm = jnp.where(oh, m, 0.0)
  sums = jnp.dot(ohw, seq, preferred_element_type=jnp.float32)
  counts = jnp.sum(ohm, axis=1, keepdims=True)
  n_unique = jnp.maximum(jnp.sum(counts), 1.0)
  vec1 = jnp.sum(sums, axis=0, keepdims=True) / n_unique
  cent = jnp.concatenate(
      [cent_ref[...], jnp.zeros((KP - K, D), jnp.float32)], axis=0)
  matrix = sums - counts * cent
  mnm = matrix - jnp.mean(matrix, axis=1, keepdims=True)
  e1 = lax.dot_general(mnm, ri_ref[...], (((0,), (1,)), ((), ())),
                       preferred_element_type=jnp.float32)
  e2 = lax.dot_general(mnm, rj_ref[...], (((0,), (1,)), ((), ())),
                       preferred_element_type=jnp.float32)
  v2 = jnp.sum(e1 * e2, axis=0, keepdims=True)
  inv = lax.rsqrt(jnp.sum(v2 * v2))
  e = jnp.concatenate([vec1, v2 * inv], axis=1)
  s = svd_ref[...]
  dd = jnp.sum(e * s)
  return e - dd * s


def _tc_body(seq_ref, w_ref, seg_ref, cent_ref, ri_ref, rj_ref, svd_ref,
             out_ref):
  i = pl.program_id(0)
  for r in range(NB):
    b = i * NB + r
    res = _tc_one_batch(seq_ref[r], w_ref[pl.ds(b, 1), :],
                        seg_ref[pl.ds(b, 1), :],
                        cent_ref, ri_ref, rj_ref, svd_ref)
    out_ref[pl.ds(b, 1), :] = res


def _tc_pool(seq, w2d, seg2d, cent32, ri, rj, svd, interpret=False):
  return pl.pallas_call(
      _tc_body,
      grid=(B // NB,),
      in_specs=[
          pl.BlockSpec((NB, L, D), lambda i: (i, 0, 0)),
          pl.BlockSpec((B, L), lambda i: (0, 0)),
          pl.BlockSpec((B, L), lambda i: (0, 0)),
          pl.BlockSpec((K, D), lambda i: (0, 0)),
          pl.BlockSpec((NT, KP), lambda i: (0, 0)),
          pl.BlockSpec((NT, KP), lambda i: (0, 0)),
          pl.BlockSpec((1, D_OUT), lambda i: (0, 0)),
      ],
      out_specs=pl.BlockSpec((B, D_OUT), lambda i: (0, 0)),
      out_shape=jax.ShapeDtypeStruct((B, D_OUT), jnp.float32),
      interpret=interpret,
  )(seq, w2d, seg2d, cent32, ri, rj, svd)


def kernel(sequence_output, input_ids, padding_mask, token_weights,
           centroids, token_to_cluster, svd_components):
  w2d, seg2d = _sc_tokens(input_ids, padding_mask, token_weights,
                          token_to_cluster)
  return _tc_pool(sequence_output, w2d, seg2d, centroids,
                  jnp.asarray(_RI), jnp.asarray(_RJ), svd_components)


# speedup vs baseline: 1.0100x; 1.0100x over previous
import functools
import math

import jax
import jax.numpy as jnp
import numpy as np
from jax import lax
from jax.experimental import pallas as pl
from jax.experimental.pallas import tpu as pltpu
from jax.experimental.pallas import tpu_sc as plsc

B, L, D = 16, 2048, 128
V = 100000
K = 30
KP = 32
D_OUT = D + (K * (K + 1)) // 2
LANES = 16
NCHUNK = L // LANES
GATHER_ROWS = 16
GATHER_W = L // GATHER_ROWS

_IU_I, _IU_J = np.triu_indices(K)


def _sc_body(ids_hbm, pm_hbm, tw_hbm, t2c_hbm, w_out, seg_out,
             idsf, pmf, twgf, clgf, table, wf, segf, sem_tw, sem_cl):
  wid = lax.axis_index("s") * 2 + lax.axis_index("c")

  @pl.when(wid < B)
  def _():
    b = wid
    pltpu.sync_copy(ids_hbm.at[b], idsf)
    pltpu.sync_copy(pm_hbm.at[b], pmf)

    copies = []
    for j in range(GATHER_ROWS):
      sl = pl.ds(j * GATHER_W, GATHER_W)
      copies.append(pltpu.async_copy(
          tw_hbm.at[idsf.at[sl]], twgf.at[sl], sem_tw))
      copies.append(pltpu.async_copy(
          t2c_hbm.at[idsf.at[sl]], clgf.at[sl], sem_cl))

    lane = lax.iota(jnp.int32, LANES)

    def scatter_body(c, _):
      sl = pl.ds(c * LANES, LANES)
      id16 = idsf[sl]
      valid = pmf[sl] != 0
      pos16 = c * LANES + lane
      _, lastm = plsc.scan_count(id16, mask=valid)
      plsc.store_scatter(table, [id16], pos16, mask=lastm & valid)
      return 0

    lax.fori_loop(0, NCHUNK, scatter_body, 0, unroll=4)

    for cp in copies:
      cp.wait()

    def check_body(c, _):
      sl = pl.ds(c * LANES, LANES)
      id16 = idsf[sl]
      valid = pmf[sl] != 0
      pos16 = c * LANES + lane
      lastpos = plsc.load_gather(table, [id16])
      is_last = valid & (lastpos == pos16)
      wf[sl] = jnp.where(is_last, twgf[sl], 0.0)
      segf[sl] = jnp.where(is_last, clgf[sl], K)
      return 0

    lax.fori_loop(0, NCHUNK, check_body, 0, unroll=4)

    pltpu.sync_copy(wf, w_out.at[b])
    pltpu.sync_copy(segf, seg_out.at[b])


def _sc_tokens(input_ids, padding_mask, token_weights, token_to_cluster):
  return pl.kernel(
      _sc_body,
      out_type=(
          jax.ShapeDtypeStruct((B, L), jnp.float32),
          jax.ShapeDtypeStruct((B, L), jnp.int32),
      ),
      mesh=plsc.VectorSubcoreMesh(core_axis_name="c", subcore_axis_name="s"),
      compiler_params=pltpu.CompilerParams(needs_layout_passes=False),
      scratch_types=[
          pltpu.VMEM((L,), jnp.int32),
          pltpu.VMEM((L,), jnp.int32),
          pltpu.VMEM((L,), jnp.float32),
          pltpu.VMEM((L,), jnp.int32),
          pltpu.VMEM((V,), jnp.int32),
          pltpu.VMEM((L,), jnp.float32),
          pltpu.VMEM((L,), jnp.int32),
          pltpu.SemaphoreType.DMA,
          pltpu.SemaphoreType.DMA,
      ],
  )(input_ids, padding_mask, token_weights, token_to_cluster)


NT = (K * (K + 1)) // 2

_RI = np.zeros((NT, KP), np.float32)
_RI[np.arange(NT), _IU_I] = np.where(_IU_I == _IU_J, 1.0, math.sqrt(2.0))
_RJ = np.zeros((NT, KP), np.float32)
_RJ[np.arange(NT), _IU_J] = 1.0


NB = 4


def _tc_one_batch(seq, w, seg, cent_ref, ri_ref, rj_ref, svd_ref):
  kidx = lax.broadcasted_iota(jnp.int32, (KP, L), 0)
  oh = seg == kidx
  ohw = jnp.where(oh, w, 0.0)
  m = (seg < K).astype(jnp.float32)
  ohm = jnp.where(oh, m, 0.0)
  sums = jnp.dot(ohw, seq, preferred_element_type=jnp.float32)
  counts = jnp.sum(ohm, axis=1, keepdims=True)
  n_unique = jnp.maximum(jnp.sum(counts), 1.0)
  vec1 = jnp.sum(sums, axis=0, keepdims=True) / n_unique
  cent = jnp.concatenate(
      [cent_ref[...], jnp.zeros((KP - K, D), jnp.float32)], axis=0)
  matrix = sums - counts * cent
  mnm = matrix - jnp.mean(matrix, axis=1, keepdims=True)
  e1 = lax.dot_general(mnm, ri_ref[...], (((0,), (1,)), ((), ())),
                       preferred_element_type=jnp.float32)
  e2 = lax.dot_general(mnm, rj_ref[...], (((0,), (1,)), ((), ())),
                       preferred_element_type=jnp.float32)
  v2 = jnp.sum(e1 * e2, axis=0, keepdims=True)
  inv = lax.rsqrt(jnp.sum(v2 * v2))
  e = jnp.concatenate([vec1, v2 * inv], axis=1)
  s = svd_ref[...]
  dd = jnp.sum(e * s)
  return e - dd * s


def _tc_body(seq_ref, w_ref, seg_ref, cent_ref, ri_ref, rj_ref, svd_ref,
             out_ref):
  i = pl.program_id(0)
  for r in range(NB):
    b = i * NB + r
    res = _tc_one_batch(seq_ref[r], w_ref[pl.ds(b, 1), :],
                        seg_ref[pl.ds(b, 1), :],
                        cent_ref, ri_ref, rj_ref, svd_ref)
    out_ref[pl.ds(b, 1), :] = res


def _tc_pool(seq, w2d, seg2d, cent32, ri, rj, svd, interpret=False):
  return pl.pallas_call(
      _tc_body,
      grid=(B // NB,),
      in_specs=[
          pl.BlockSpec((NB, L, D), lambda i: (i, 0, 0)),
          pl.BlockSpec((B, L), lambda i: (0, 0)),
          pl.BlockSpec((B, L), lambda i: (0, 0)),
          pl.BlockSpec((K, D), lambda i: (0, 0)),
          pl.BlockSpec((NT, KP), lambda i: (0, 0)),
          pl.BlockSpec((NT, KP), lambda i: (0, 0)),
          pl.BlockSpec((1, D_OUT), lambda i: (0, 0)),
      ],
      out_specs=pl.BlockSpec((B, D_OUT), lambda i: (0, 0)),
      out_shape=jax.ShapeDtypeStruct((B, D_OUT), jnp.float32),
      interpret=interpret,
  )(seq, w2d, seg2d, cent32, ri, rj, svd)


def kernel(sequence_output, input_ids, padding_mask, token_weights,
           centroids, token_to_cluster, svd_components):
  w2d, seg2d = _sc_tokens(input_ids, padding_mask, token_weights,
                          token_to_cluster)
  return _tc_pool(sequence_output, w2d, seg2d, centroids,
                  jnp.asarray(_RI), jnp.asarray(_RJ), svd_components)
